# baseline (speedup 1.0000x reference)
import jax
import jax.numpy as jnp
from jax import lax
from jax.experimental import pallas as pl
from jax.experimental.pallas import tpu as pltpu
from jax.experimental.pallas import tpu_sc as plsc

N = 10000
D = 128
E = 320000
P = 16
OUT = 128

NW = 32
NH = 5
E2 = E // NH
EPW = E2 // NW
CHUNK = 200
NB_PREP = 10
BE = 2560


def _prep_body(x_ref, wcat_ref, bcat_ref, wh1_ref, wh2_ref,
               t1_ref, t2_ref):
    xb = x_ref[...]
    z = jnp.maximum(
        jnp.dot(xb, wcat_ref[...], preferred_element_type=jnp.float32)
        + bcat_ref[...], 0.0)
    h1 = jnp.dot(z, wh1_ref[...], preferred_element_type=jnp.float32)
    h2 = jnp.dot(z, wh2_ref[...], preferred_element_type=jnp.float32)
    z4 = jnp.sqrt(z[:, 48:64])
    rb = z.shape[0]

    def pk(a):
        half = a.shape[1] // 2
        lo = lax.bitcast_convert_type(
            a[:, :half].astype(jnp.bfloat16), jnp.uint16).astype(jnp.uint32)
        hi = lax.bitcast_convert_type(
            a[:, half:].astype(jnp.bfloat16), jnp.uint16).astype(jnp.uint32)
        return lax.bitcast_convert_type(lo | (hi << 16), jnp.int32)

    n2 = jnp.sum(xb * xb, axis=1, keepdims=True)
    xh = xb / jnp.maximum(jnp.sqrt(n2), 1e-8)
    xq = lax.bitcast_convert_type(
        xh.astype(jnp.float8_e4m3fn), jnp.uint8).astype(jnp.uint32)
    xw = lax.bitcast_convert_type(
        xq[:, 0:32] | (xq[:, 32:64] << 8) | (xq[:, 64:96] << 16)
        | (xq[:, 96:128] << 24), jnp.int32)
    pad = jnp.zeros((rb, 24), jnp.int32)
    t1_ref[...] = jnp.concatenate([pk(h1), pk(z4), xw, pad], axis=1)
    t2_ref[...] = jnp.concatenate([pk(h2), pk(z4), xw, pad], axis=1)


def _prep_call(x, wcat, bcat, wh1, wh2):
    rb = N // NB_PREP
    return pl.pallas_call(
        _prep_body,
        grid=(NB_PREP,),
        in_specs=[
            pl.BlockSpec((rb, D), lambda i: (i, 0)),
            pl.BlockSpec((D, 64), lambda i: (0, 0)),
            pl.BlockSpec((1, 64), lambda i: (0, 0)),
            pl.BlockSpec((64, OUT), lambda i: (0, 0)),
            pl.BlockSpec((64, OUT), lambda i: (0, 0)),
        ],
        out_specs=[
            pl.BlockSpec((rb, D), lambda i: (i, 0)),
            pl.BlockSpec((rb, D), lambda i: (i, 0)),
        ],
        out_shape=[
            jax.ShapeDtypeStruct((N, D), jnp.int32),
            jax.ShapeDtypeStruct((N, D), jnp.int32),
        ],
    )(x, wcat, bcat, wh1, wh2)


def _sc_body(t1, t2, rowv, colv,
             ga, gb,
             idxr, idxc, bufa, bufb, sem):
    cid = lax.axis_index("c")
    sid = lax.axis_index("s")
    wid = sid * 2 + cid
    base0 = wid * EPW

    def chunk_body(ci, carry):
        base = base0 + ci * CHUNK
        pltpu.sync_copy(rowv.at[pl.ds(base, CHUNK)], idxr)
        pltpu.sync_copy(colv.at[pl.ds(base, CHUNK)], idxc)
        d1 = pltpu.async_copy(t1.at[idxr], bufa, sem)
        d2 = pltpu.async_copy(t2.at[idxc], bufb, sem)
        d1.wait()
        d2.wait()
        pltpu.sync_copy(bufa, ga.at[pl.ds(base, CHUNK)])
        pltpu.sync_copy(bufb, gb.at[pl.ds(base, CHUNK)])
        return carry

    lax.fori_loop(0, EPW // CHUNK, chunk_body, 0)


def _sc_call(t1, t2, rowv, colv):
    mesh = plsc.VectorSubcoreMesh(core_axis_name="c", subcore_axis_name="s")
    fn = pl.kernel(
        _sc_body,
        out_type=[
            jax.ShapeDtypeStruct((E2, D), jnp.int32),
            jax.ShapeDtypeStruct((E2, D), jnp.int32),
        ],
        mesh=mesh,
        scratch_types=[
            pltpu.VMEM((CHUNK,), jnp.int32),
            pltpu.VMEM((CHUNK,), jnp.int32),
            pltpu.VMEM((CHUNK, D), jnp.int32),
            pltpu.VMEM((CHUNK, D), jnp.int32),
            pltpu.SemaphoreType.DMA,
        ],
    )
    return fn(t1, t2, rowv, colv)


def _lo(w):
    return lax.bitcast_convert_type(lax.shift_left(w, 16), jnp.float32)


def _hi(w):
    return lax.bitcast_convert_type(w & jnp.int32(-65536), jnp.float32)


def _f8wide(w):
    m = jnp.int32(0xFF)
    parts = [w & m,
             lax.shift_right_logical(w, 8) & m,
             lax.shift_right_logical(w, 16) & m,
             lax.shift_right_logical(w, 24)]
    wide = jnp.concatenate(parts, axis=1)
    return lax.bitcast_convert_type(
        wide.astype(jnp.uint8), jnp.float8_e4m3fn).astype(jnp.float32)


def _final_body(prev_ref, ga_ref, gb_ref, eat_ref,
                dw_ref, fw_ref, wfs_ref, bf_ref, out_ref):
    del prev_ref
    ga = ga_ref[...]
    gb = gb_ref[...]
    dw = dw_ref[...]
    fw = fw_ref[...]
    wfs = wfs_ref[...]
    bf = bf_ref[...]
    eat = eat_ref[...]
    gaz = ga[:, 64:72]
    gbz = gb[:, 64:72]
    q_lo = _lo(gaz) * _lo(gbz)
    q_hi = _hi(gaz) * _hi(gbz)
    xr = _f8wide(ga[:, 72:104])
    xc = _f8wide(gb[:, 72:104])
    psum = xr * xc
    for half, sl in ((0, slice(0, 64)), (1, slice(64, 128))):
        ext = _lo if half == 0 else _hi
        acc = ext(ga[:, 0:64]) + ext(gb[:, 0:64]) + bf[:, sl]
        acc = acc + jnp.dot(q_lo, dw[0:8, sl],
                            preferred_element_type=jnp.float32)
        acc = acc + jnp.dot(q_hi, dw[8:16, sl],
                            preferred_element_type=jnp.float32)
        acc = acc + lax.dot_general(eat, fw[:, sl], (((0,), (0,)), ((), ())),
                                    preferred_element_type=jnp.float32)
        wcos = jnp.broadcast_to(wfs[:, sl], (D, 64))
        acc = acc + jnp.dot(psum, wcos, preferred_element_type=jnp.float32)
        out_ref[:, sl] = jnp.maximum(acc, 0.0)


def _final_call(prev, ga, gb, eat, dw, fw, wfs, bfv, h):
    nbh = E2 // BE
    body = _final_body if prev is not None else (
        lambda *refs: _final_body(None, *refs))
    in_specs = [
        pl.BlockSpec((BE, D), lambda i: (i, 0)),
        pl.BlockSpec((BE, D), lambda i: (i, 0)),
        pl.BlockSpec((P, BE), lambda i, _h=h: (0, i + _h * (E2 // BE))),
        pl.BlockSpec((P, OUT), lambda i: (0, 0)),
        pl.BlockSpec((P, OUT), lambda i: (0, 0)),
        pl.BlockSpec((1, OUT), lambda i: (0, 0)),
        pl.BlockSpec((1, OUT), lambda i: (0, 0)),
    ]
    args = (ga, gb, eat, dw, fw, wfs, bfv)
    aliases = {}
    if prev is not None:
        in_specs = [pl.BlockSpec((8, OUT), lambda i: (0, 0))] + in_specs
        args = (prev,) + args
        aliases = {0: 0}
    return pl.pallas_call(
        body,
        grid=(nbh,),
        in_specs=in_specs,
        out_specs=pl.BlockSpec((BE, OUT),
                               lambda i, _h=h: (i + _h * (E2 // BE), 0)),
        out_shape=jax.ShapeDtypeStruct((E, OUT), jnp.float32),
        input_output_aliases=aliases,
    )(*args)


def kernel(x, edge_index, edge_attr, W1, b1, W2, b2, W3, b3, W4, b4, Wf, bf):
    row = edge_index[0].astype(jnp.int32)
    col = edge_index[1].astype(jnp.int32)
    eat = edge_attr.T
    wcat = jnp.concatenate([W1, W2, W3, W4], axis=1)
    bcat = jnp.concatenate([b1, b2, b3, b4]).reshape(1, 4 * P)
    A = Wf[0:P]
    Bw = Wf[P:2 * P]
    Cw = Wf[2 * P:3 * P]
    Dw = Wf[3 * P:4 * P]
    wfs = Wf[4 * P:4 * P + 1]
    Fw = Wf[4 * P + 1:]
    wh1 = jnp.zeros((4 * P, OUT), jnp.float32).at[0:P].set(A).at[2 * P:3 * P].set(Cw)
    wh2 = jnp.zeros((4 * P, OUT), jnp.float32).at[P:2 * P].set(Bw).at[2 * P:3 * P].set(-Cw)
    t1, t2 = _prep_call(x, wcat, bcat, wh1, wh2)
    bfv = bf.reshape(1, OUT)
    parts = []
    for h in range(NH):
        sl = slice(h * E2, (h + 1) * E2)
        parts.append(_sc_call(t1, t2, row[sl], col[sl]))
    out = None
    for h in range(NH):
        ga, gb = parts[h]
        out = _final_call(out, ga, gb, eat, Dw, Fw, wfs, bfv, h)
    return out


# speedup vs baseline: 1.3492x; 1.3492x over previous
import jax
import jax.numpy as jnp
from jax import lax
from jax.experimental import pallas as pl
from jax.experimental.pallas import tpu as pltpu
from jax.experimental.pallas import tpu_sc as plsc

N = 10000
D = 128
E = 320000
P = 16
OUT = 128

NW = 32
NH = 5
E2 = E // NH
EPW = E2 // NW
CHUNK = 200
NB_PREP = 10
BE = 2560


def _prep_body(x_ref, wcat_ref, bcat_ref, wh1_ref, wh2_ref,
               t1_ref, t2_ref):
    xb = x_ref[...]
    z = jnp.maximum(
        jnp.dot(xb, wcat_ref[...], preferred_element_type=jnp.float32)
        + bcat_ref[...], 0.0)
    h1 = jnp.dot(z, wh1_ref[...], preferred_element_type=jnp.float32)
    h2 = jnp.dot(z, wh2_ref[...], preferred_element_type=jnp.float32)
    z4 = jnp.sqrt(z[:, 48:64])
    rb = z.shape[0]

    def pk(a):
        half = a.shape[1] // 2
        lo = lax.bitcast_convert_type(
            a[:, :half].astype(jnp.bfloat16), jnp.uint16).astype(jnp.uint32)
        hi = lax.bitcast_convert_type(
            a[:, half:].astype(jnp.bfloat16), jnp.uint16).astype(jnp.uint32)
        return lax.bitcast_convert_type(lo | (hi << 16), jnp.int32)

    n2 = jnp.sum(xb * xb, axis=1, keepdims=True)
    xh = xb / jnp.maximum(jnp.sqrt(n2), 1e-8)
    xq = lax.bitcast_convert_type(
        xh.astype(jnp.float8_e4m3fn), jnp.uint8).astype(jnp.uint32)
    xw = lax.bitcast_convert_type(
        xq[:, 0:32] | (xq[:, 32:64] << 8) | (xq[:, 64:96] << 16)
        | (xq[:, 96:128] << 24), jnp.int32)
    pad = jnp.zeros((rb, 24), jnp.int32)
    t1_ref[...] = jnp.concatenate([pk(h1), pk(z4), xw, pad], axis=1)
    t2_ref[...] = jnp.concatenate([pk(h2), pk(z4), xw, pad], axis=1)


def _prep_call(x, wcat, bcat, wh1, wh2):
    rb = N // NB_PREP
    return pl.pallas_call(
        _prep_body,
        grid=(NB_PREP,),
        in_specs=[
            pl.BlockSpec((rb, D), lambda i: (i, 0)),
            pl.BlockSpec((D, 64), lambda i: (0, 0)),
            pl.BlockSpec((1, 64), lambda i: (0, 0)),
            pl.BlockSpec((64, OUT), lambda i: (0, 0)),
            pl.BlockSpec((64, OUT), lambda i: (0, 0)),
        ],
        out_specs=[
            pl.BlockSpec((rb, D), lambda i: (i, 0)),
            pl.BlockSpec((rb, D), lambda i: (i, 0)),
        ],
        out_shape=[
            jax.ShapeDtypeStruct((N, D), jnp.int32),
            jax.ShapeDtypeStruct((N, D), jnp.int32),
        ],
    )(x, wcat, bcat, wh1, wh2)


def _sc_body(t1, t2, rowv, colv,
             ga, gb,
             idxr, idxc, bufa, bufb, sem):
    cid = lax.axis_index("c")
    sid = lax.axis_index("s")
    wid = sid * 2 + cid
    base0 = wid * EPW

    def chunk_body(ci, carry):
        base = base0 + ci * CHUNK
        pltpu.sync_copy(rowv.at[pl.ds(base, CHUNK)], idxr)
        pltpu.sync_copy(colv.at[pl.ds(base, CHUNK)], idxc)
        d1 = pltpu.async_copy(t1.at[idxr], bufa, sem)
        d2 = pltpu.async_copy(t2.at[idxc], bufb, sem)
        d1.wait()
        d2.wait()
        pltpu.sync_copy(bufa, ga.at[pl.ds(base, CHUNK)])
        pltpu.sync_copy(bufb, gb.at[pl.ds(base, CHUNK)])
        return carry

    lax.fori_loop(0, EPW // CHUNK, chunk_body, 0)


def _sc_call(t1, t2, rowv, colv):
    mesh = plsc.VectorSubcoreMesh(core_axis_name="c", subcore_axis_name="s")
    fn = pl.kernel(
        _sc_body,
        out_type=[
            jax.ShapeDtypeStruct((E2, D), jnp.int32),
            jax.ShapeDtypeStruct((E2, D), jnp.int32),
        ],
        mesh=mesh,
        scratch_types=[
            pltpu.VMEM((CHUNK,), jnp.int32),
            pltpu.VMEM((CHUNK,), jnp.int32),
            pltpu.VMEM((CHUNK, D), jnp.int32),
            pltpu.VMEM((CHUNK, D), jnp.int32),
            pltpu.SemaphoreType.DMA,
        ],
    )
    return fn(t1, t2, rowv, colv)


def _lo(w):
    return lax.bitcast_convert_type(lax.shift_left(w, 16), jnp.float32)


def _hi(w):
    return lax.bitcast_convert_type(w & jnp.int32(-65536), jnp.float32)


def _f8(w, k):
    b = lax.shift_right_logical(w, 8 * k) & jnp.int32(0xFF)
    return lax.bitcast_convert_type(
        b.astype(jnp.uint8), jnp.float8_e4m3fn).astype(jnp.float32)


def _final_body(prev_ref, ga_ref, gb_ref, eat_ref,
                dw_ref, fw_ref, wfs_ref, bf_ref, out_ref):
    del prev_ref
    ga = ga_ref[...]
    gb = gb_ref[...]
    dw = dw_ref[...]
    fw = fw_ref[...]
    wfs = wfs_ref[...]
    bf = bf_ref[...]
    eat = eat_ref[...]
    gaz = ga[:, 64:72]
    gbz = gb[:, 64:72]
    q_lo = _lo(gaz) * _lo(gbz)
    q_hi = _hi(gaz) * _hi(gbz)
    gax = ga[:, 72:104]
    gbx = gb[:, 72:104]
    prods = [_f8(gax, k) * _f8(gbx, k) for k in range(4)]
    psum = (prods[0] + prods[1]) + (prods[2] + prods[3])
    for half, sl in ((0, slice(0, 64)), (1, slice(64, 128))):
        ext = _lo if half == 0 else _hi
        acc = ext(ga[:, 0:64]) + ext(gb[:, 0:64]) + bf[:, sl]
        acc = acc + jnp.dot(q_lo, dw[0:8, sl],
                            preferred_element_type=jnp.float32)
        acc = acc + jnp.dot(q_hi, dw[8:16, sl],
                            preferred_element_type=jnp.float32)
        acc = acc + lax.dot_general(eat, fw[:, sl], (((0,), (0,)), ((), ())),
                                    preferred_element_type=jnp.float32)
        wcos = jnp.broadcast_to(wfs[:, sl], (32, 64))
        acc = acc + jnp.dot(psum, wcos, preferred_element_type=jnp.float32)
        out_ref[:, sl] = jnp.maximum(acc, 0.0)


def _final_call(prev, ga, gb, eat, dw, fw, wfs, bfv, h):
    nbh = E2 // BE
    body = _final_body if prev is not None else (
        lambda *refs: _final_body(None, *refs))
    in_specs = [
        pl.BlockSpec((BE, D), lambda i: (i, 0)),
        pl.BlockSpec((BE, D), lambda i: (i, 0)),
        pl.BlockSpec((P, BE), lambda i, _h=h: (0, i + _h * (E2 // BE))),
        pl.BlockSpec((P, OUT), lambda i: (0, 0)),
        pl.BlockSpec((P, OUT), lambda i: (0, 0)),
        pl.BlockSpec((1, OUT), lambda i: (0, 0)),
        pl.BlockSpec((1, OUT), lambda i: (0, 0)),
    ]
    args = (ga, gb, eat, dw, fw, wfs, bfv)
    aliases = {}
    if prev is not None:
        in_specs = [pl.BlockSpec((8, OUT), lambda i: (0, 0))] + in_specs
        args = (prev,) + args
        aliases = {0: 0}
    return pl.pallas_call(
        body,
        grid=(nbh,),
        in_specs=in_specs,
        out_specs=pl.BlockSpec((BE, OUT),
                               lambda i, _h=h: (i + _h * (E2 // BE), 0)),
        out_shape=jax.ShapeDtypeStruct((E, OUT), jnp.float32),
        input_output_aliases=aliases,
    )(*args)


def kernel(x, edge_index, edge_attr, W1, b1, W2, b2, W3, b3, W4, b4, Wf, bf):
    row = edge_index[0].astype(jnp.int32)
    col = edge_index[1].astype(jnp.int32)
    eat = edge_attr.T
    wcat = jnp.concatenate([W1, W2, W3, W4], axis=1)
    bcat = jnp.concatenate([b1, b2, b3, b4]).reshape(1, 4 * P)
    A = Wf[0:P]
    Bw = Wf[P:2 * P]
    Cw = Wf[2 * P:3 * P]
    Dw = Wf[3 * P:4 * P]
    wfs = Wf[4 * P:4 * P + 1]
    Fw = Wf[4 * P + 1:]
    wh1 = jnp.zeros((4 * P, OUT), jnp.float32).at[0:P].set(A).at[2 * P:3 * P].set(Cw)
    wh2 = jnp.zeros((4 * P, OUT), jnp.float32).at[P:2 * P].set(Bw).at[2 * P:3 * P].set(-Cw)
    t1, t2 = _prep_call(x, wcat, bcat, wh1, wh2)
    bfv = bf.reshape(1, OUT)
    parts = []
    for h in range(NH):
        sl = slice(h * E2, (h + 1) * E2)
        parts.append(_sc_call(t1, t2, row[sl], col[sl]))
    out = None
    for h in range(NH):
        ga, gb = parts[h]
        out = _final_call(out, ga, gb, eat, Dw, Fw, wfs, bfv, h)
    return out
